# baseline (speedup 1.0000x reference)
import jax
import jax.numpy as jnp
from jax import lax
from jax.experimental import pallas as pl
from jax.experimental.pallas import tpu as pltpu
from jax.experimental.pallas import tpu_sc as plsc

VOCAB = 1000000
DIM = 16
BATCH = 16384

NC = 2
NS = 16
NW = NC * NS
LANES = 16

SUP = 8
B_PER_W = BATCH // NW
CHUNK = 128
N_CHUNKS = B_PER_W // CHUNK
GROUPS_PER_CHUNK = CHUNK // LANES


def _sc_body(u_hbm, v_hbm, table_hbm, out_hbm,
             idx_u, idx_v, sup_u, sup_v, u_sup, v_sup, out_loc, sem):
    wid = lax.axis_index("s") * NC + lax.axis_index("c")
    base = wid * B_PER_W

    for c in range(N_CHUNKS):
        pltpu.sync_copy(u_hbm.at[pl.ds(base + c * CHUNK, CHUNK)], idx_u.at[c])
        pltpu.sync_copy(v_hbm.at[pl.ds(base + c * CHUNK, CHUNK)], idx_v.at[c])

    def sup_slice(k, _):
        o = k * LANES
        sup_u[pl.ds(o, LANES)] = jnp.right_shift(
            idx_u.at[k // GROUPS_PER_CHUNK][pl.ds((k % GROUPS_PER_CHUNK) * LANES, LANES)], 3)
        sup_v[pl.ds(o, LANES)] = jnp.right_shift(
            idx_v.at[k // GROUPS_PER_CHUNK][pl.ds((k % GROUPS_PER_CHUNK) * LANES, LANES)], 3)
        return _

    lax.fori_loop(0, N_CHUNKS * GROUPS_PER_CHUNK, sup_slice, None)

    lane = lax.iota(jnp.int32, LANES)

    def compute_chunk(c, ub, vb):
        def group(g, _):
            o = g * LANES
            mu = idx_u.at[c][pl.ds(o, LANES)] & 7
            mv = idx_v.at[c][pl.ds(o, LANES)] & 7
            acc = jnp.zeros((LANES,), jnp.float32)
            rows = lane + o
            cu = mu * DIM
            cv = mv * DIM
            for d in range(DIM):
                uu = plsc.load_gather(ub, [rows, cu + d])
                vv = plsc.load_gather(vb, [rows, cv + d])
                acc = acc + uu * vv
            sig = 1.0 / (1.0 + jnp.exp(-acc))
            out_loc[pl.ds(c * CHUNK + o, LANES)] = sig
            return _

        lax.fori_loop(0, GROUPS_PER_CHUNK, group, None)

    def fire(c, ub, vb):
        cu = pltpu.async_copy(
            table_hbm.at[sup_u.at[pl.ds(c * CHUNK, CHUNK)]], ub, sem)
        cv = pltpu.async_copy(
            table_hbm.at[sup_v.at[pl.ds(c * CHUNK, CHUNK)]], vb, sem)
        return cu, cv

    bufs = [(u_sup.at[0], v_sup.at[0]), (u_sup.at[1], v_sup.at[1])]
    pending = fire(0, *bufs[0])
    for c in range(N_CHUNKS):
        for cp in pending:
            cp.wait()
        if c + 1 < N_CHUNKS:
            nxt = fire(c + 1, *bufs[(c + 1) % 2])
        compute_chunk(c, *bufs[c % 2])
        if c + 1 < N_CHUNKS:
            pending = nxt

    pltpu.sync_copy(out_loc, out_hbm.at[pl.ds(base, B_PER_W)])


@jax.jit
def kernel(u, v, embed):
    mesh = plsc.VectorSubcoreMesh(
        core_axis_name="c", subcore_axis_name="s",
        num_cores=NC, num_subcores=NS,
    )
    k = pl.kernel(
        _sc_body,
        out_type=jax.ShapeDtypeStruct((BATCH,), jnp.float32),
        mesh=mesh,
        scratch_types=[
            pltpu.VMEM((N_CHUNKS, CHUNK), jnp.int32),
            pltpu.VMEM((N_CHUNKS, CHUNK), jnp.int32),
            pltpu.VMEM((B_PER_W,), jnp.int32),
            pltpu.VMEM((B_PER_W,), jnp.int32),
            pltpu.VMEM((2, CHUNK, SUP * DIM), jnp.float32),
            pltpu.VMEM((2, CHUNK, SUP * DIM), jnp.float32),
            pltpu.VMEM((B_PER_W,), jnp.float32),
            pltpu.SemaphoreType.DMA,
        ],
        compiler_params=pltpu.CompilerParams(needs_layout_passes=False),
    )
    table3 = embed.reshape(VOCAB // SUP, SUP * DIM)
    return k(u.astype(jnp.int32), v.astype(jnp.int32), table3)
